# baseline (speedup 1.0000x reference)
import functools

import jax
import jax.numpy as jnp
from jax import lax
from jax.experimental import pallas as pl
from jax.experimental.pallas import tpu as pltpu
from jax.experimental.pallas import tpu_sc as plsc

B = 16384
F = 128
NC = 2
NS = 16
L = 16
NW = NC * NS
BPW = B // NW
CH = 128
NCH = BPW // CH


def _sc_body(user_hbm, item_hbm, uf_hbm, if_hbm, ub_hbm, ib_hbm, out_hbm,
             idx_u, idx_i, uf_v, if_v, ub_v, ib_v, out_v,
             gsem0, gsem1, osem):
    cid = lax.axis_index("c")
    sid = lax.axis_index("s")
    wid = sid * NC + cid
    base = wid * BPW
    gsems = [gsem0, gsem1]

    pltpu.sync_copy(user_hbm.at[pl.ds(base, BPW)], idx_u)
    pltpu.sync_copy(item_hbm.at[pl.ds(base, BPW)], idx_i)

    def fire(c):
        s = c % 2
        iu = idx_u.at[pl.ds(c * CH, CH)]
        ii = idx_i.at[pl.ds(c * CH, CH)]
        return [
            pltpu.async_copy(uf_hbm.at[iu], uf_v.at[s], gsems[s]),
            pltpu.async_copy(if_hbm.at[ii], if_v.at[s], gsems[s]),
            pltpu.async_copy(ub_hbm.at[iu], ub_v.at[s], gsems[s]),
            pltpu.async_copy(ib_hbm.at[ii], ib_v.at[s], gsems[s]),
        ]

    lane = lax.iota(jnp.int32, L)
    inflight = fire(0)
    out_cps = []
    for c in range(NCH):
        s = c % 2
        if c + 1 < NCH:
            nxt = fire(c + 1)
        for cp in inflight:
            cp.wait()
        if c + 1 < NCH:
            inflight = nxt
        @plsc.parallel_loop(0, CH // L, unroll=2)
        def gloop(g):

            def rbody(k, res):
                r = g * L + k
                acc0 = uf_v[s, r, pl.ds(0, L)] * if_v[s, r, pl.ds(0, L)]
                acc1 = uf_v[s, r, pl.ds(L, L)] * if_v[s, r, pl.ds(L, L)]
                for j in range(2, F // L, 2):
                    acc0 += (uf_v[s, r, pl.ds(j * L, L)]
                             * if_v[s, r, pl.ds(j * L, L)])
                    acc1 += (uf_v[s, r, pl.ds((j + 1) * L, L)]
                             * if_v[s, r, pl.ds((j + 1) * L, L)])
                t = jnp.sum(acc0 + acc1)
                return jnp.where(lane == k, t, res)

            res = lax.fori_loop(0, L, rbody, jnp.zeros((L,), jnp.float32))
            sgm = pl.ds(g * L, L)
            pred = res + ub_v[s, sgm] + ib_v[s, sgm]
            out_v[c, sgm] = 1.0 / (1.0 + jnp.exp(-pred))
        out_cps.append(pltpu.async_copy(
            out_v.at[c], out_hbm.at[pl.ds(base + c * CH, CH)], osem))
    for cp in out_cps:
        cp.wait()


@functools.partial(
    pl.kernel,
    mesh=plsc.VectorSubcoreMesh(core_axis_name="c", subcore_axis_name="s"),
    out_type=jax.ShapeDtypeStruct((B,), jnp.float32),
    compiler_params=pltpu.CompilerParams(
        needs_layout_passes=False, use_tc_tiling_on_sc=False),
    scratch_types=[
        pltpu.VMEM((BPW,), jnp.int32),
        pltpu.VMEM((BPW,), jnp.int32),
        pltpu.VMEM((2, CH, F), jnp.float32),
        pltpu.VMEM((2, CH, F), jnp.float32),
        pltpu.VMEM((2, CH), jnp.float32),
        pltpu.VMEM((2, CH), jnp.float32),
        pltpu.VMEM((NCH, CH), jnp.float32),
        pltpu.SemaphoreType.DMA,
        pltpu.SemaphoreType.DMA,
        pltpu.SemaphoreType.DMA,
    ],
)
def _svdpp_sc(user_hbm, item_hbm, uf_hbm, if_hbm, ub_hbm, ib_hbm, out_hbm,
              idx_u, idx_i, uf_v, if_v, ub_v, ib_v, out_v,
              gsem0, gsem1, osem):
    _sc_body(user_hbm, item_hbm, uf_hbm, if_hbm, ub_hbm, ib_hbm, out_hbm,
             idx_u, idx_i, uf_v, if_v, ub_v, ib_v, out_v, gsem0, gsem1, osem)


def kernel(user, item, user_factors, item_factors, user_biases, item_biases,
           user_implicit):
    del user_implicit
    ub = jnp.pad(user_biases, ((0, -user_biases.shape[0] % 1024), (0, 0)))
    ib = jnp.pad(item_biases, ((0, -item_biases.shape[0] % 1024), (0, 0)))
    return _svdpp_sc(user, item, user_factors, item_factors,
                     ub.reshape(-1), ib.reshape(-1))


# speedup vs baseline: 1.0064x; 1.0064x over previous
import functools

import jax
import jax.numpy as jnp
from jax import lax
from jax.experimental import pallas as pl
from jax.experimental.pallas import tpu as pltpu
from jax.experimental.pallas import tpu_sc as plsc

B = 16384
F = 128
NC = 2
NS = 16
L = 16
NW = NC * NS
BPW = B // NW
CH = 128
NCH = BPW // CH


def _sc_body(user_hbm, item_hbm, uf_hbm, if_hbm, ub_hbm, ib_hbm, out_hbm,
             idx_u, idx_i, uf_v, if_v, ub_v, ib_v, out_v,
             gsem0, gsem1, osem):
    cid = lax.axis_index("c")
    sid = lax.axis_index("s")
    wid = sid * NC + cid
    base = wid * BPW
    gsems = [gsem0, gsem1]

    pltpu.sync_copy(user_hbm.at[pl.ds(base, BPW)], idx_u)
    pltpu.sync_copy(item_hbm.at[pl.ds(base, BPW)], idx_i)

    def fire(c):
        s = c % 2
        iu = idx_u.at[pl.ds(c * CH, CH)]
        ii = idx_i.at[pl.ds(c * CH, CH)]
        return [
            pltpu.async_copy(uf_hbm.at[iu], uf_v.at[s], gsems[s]),
            pltpu.async_copy(if_hbm.at[ii], if_v.at[s], gsems[s]),
            pltpu.async_copy(ub_hbm.at[iu], ub_v.at[s], gsems[s]),
            pltpu.async_copy(ib_hbm.at[ii], ib_v.at[s], gsems[s]),
        ]

    lane = lax.iota(jnp.int32, L)
    inflight = fire(0)
    out_cps = []
    for c in range(NCH):
        s = c % 2
        if c + 1 < NCH:
            nxt = fire(c + 1)
        for cp in inflight:
            cp.wait()
        if c + 1 < NCH:
            inflight = nxt
        @plsc.parallel_loop(0, CH // L)
        def gloop(g):

            def rbody(k, res):
                r = g * L + k
                acc0 = uf_v[s, r, pl.ds(0, L)] * if_v[s, r, pl.ds(0, L)]
                acc1 = uf_v[s, r, pl.ds(L, L)] * if_v[s, r, pl.ds(L, L)]
                for j in range(2, F // L, 2):
                    acc0 += (uf_v[s, r, pl.ds(j * L, L)]
                             * if_v[s, r, pl.ds(j * L, L)])
                    acc1 += (uf_v[s, r, pl.ds((j + 1) * L, L)]
                             * if_v[s, r, pl.ds((j + 1) * L, L)])
                t = jnp.sum(acc0 + acc1)
                return jnp.where(lane == k, t, res)

            res = lax.fori_loop(0, L, rbody, jnp.zeros((L,), jnp.float32))
            sgm = pl.ds(g * L, L)
            pred = res + ub_v[s, sgm] + ib_v[s, sgm]
            out_v[c, sgm] = 1.0 / (1.0 + jnp.exp(-pred))
        out_cps.append(pltpu.async_copy(
            out_v.at[c], out_hbm.at[pl.ds(base + c * CH, CH)], osem))
    for cp in out_cps:
        cp.wait()


@functools.partial(
    pl.kernel,
    mesh=plsc.VectorSubcoreMesh(core_axis_name="c", subcore_axis_name="s"),
    out_type=jax.ShapeDtypeStruct((B,), jnp.float32),
    compiler_params=pltpu.CompilerParams(
        needs_layout_passes=False, use_tc_tiling_on_sc=False),
    scratch_types=[
        pltpu.VMEM((BPW,), jnp.int32),
        pltpu.VMEM((BPW,), jnp.int32),
        pltpu.VMEM((2, CH, F), jnp.float32),
        pltpu.VMEM((2, CH, F), jnp.float32),
        pltpu.VMEM((2, CH), jnp.float32),
        pltpu.VMEM((2, CH), jnp.float32),
        pltpu.VMEM((NCH, CH), jnp.float32),
        pltpu.SemaphoreType.DMA,
        pltpu.SemaphoreType.DMA,
        pltpu.SemaphoreType.DMA,
    ],
)
def _svdpp_sc(user_hbm, item_hbm, uf_hbm, if_hbm, ub_hbm, ib_hbm, out_hbm,
              idx_u, idx_i, uf_v, if_v, ub_v, ib_v, out_v,
              gsem0, gsem1, osem):
    _sc_body(user_hbm, item_hbm, uf_hbm, if_hbm, ub_hbm, ib_hbm, out_hbm,
             idx_u, idx_i, uf_v, if_v, ub_v, ib_v, out_v, gsem0, gsem1, osem)


def kernel(user, item, user_factors, item_factors, user_biases, item_biases,
           user_implicit):
    del user_implicit
    ub = jnp.pad(user_biases, ((0, -user_biases.shape[0] % 1024), (0, 0)))
    ib = jnp.pad(item_biases, ((0, -item_biases.shape[0] % 1024), (0, 0)))
    return _svdpp_sc(user, item, user_factors, item_factors,
                     ub.reshape(-1), ib.reshape(-1))
